# baseline (speedup 1.0000x reference)
import functools

import jax
import jax.numpy as jnp
from jax.experimental import pallas as pl
from jax.experimental.pallas import tpu as pltpu

KEY_DIM = 128
MEM_SIZE = 65536
CHOOSE_K = 256
FC_DIM = 1024
BATCH = 256

MEM_CHUNK = 8192
N_CHUNKS = MEM_SIZE // MEM_CHUNK


def _q_kernel(x_ref, w1_ref, b1_ref, w2_ref, b2_ref, q_ref):
    h = jnp.maximum(
        jnp.dot(x_ref[...], w1_ref[...], preferred_element_type=jnp.float32)
        + b1_ref[...], 0.0)
    q = jnp.dot(h, w2_ref[...], preferred_element_type=jnp.float32) + b2_ref[...]
    qn = jnp.sqrt(jnp.sum(q * q, axis=1, keepdims=True))
    q_ref[...] = q / (qn + 1e-8)


def _sim_kernel(q_ref, mk_ref, sim_ref, nk_ref):
    nk_ref[...] = mk_ref[...]
    sim_ref[...] = jax.lax.dot_general(
        q_ref[...], mk_ref[...], (((1,), (1,)), ((), ())),
        preferred_element_type=jnp.float32)


def kernel(x, label, W1, b1, W2, b2, mem_keys, mem_values, mem_ages):
    B = x.shape[0]
    xf = x.reshape(B, -1)

    q = pl.pallas_call(
        _q_kernel,
        out_shape=jax.ShapeDtypeStruct((B, KEY_DIM), jnp.float32),
    )(xf, W1, b1.reshape(1, FC_DIM), W2, b2.reshape(1, KEY_DIM))

    sim, new_keys_base = pl.pallas_call(
        _sim_kernel,
        grid=(N_CHUNKS,),
        in_specs=[
            pl.BlockSpec((B, KEY_DIM), lambda i: (0, 0)),
            pl.BlockSpec((MEM_CHUNK, KEY_DIM), lambda i: (i, 0)),
        ],
        out_specs=[
            pl.BlockSpec((B, MEM_CHUNK), lambda i: (0, i)),
            pl.BlockSpec((MEM_CHUNK, KEY_DIM), lambda i: (i, 0)),
        ],
        out_shape=[
            jax.ShapeDtypeStruct((B, MEM_SIZE), jnp.float32),
            jax.ShapeDtypeStruct((MEM_SIZE, KEY_DIM), jnp.float32),
        ],
    )(q, mem_keys)

    topv, topi = jax.lax.top_k(sim, CHOOSE_K)
    w = jax.nn.softmax(topv, axis=1)
    vals = jnp.take(mem_values, topi, axis=0).astype(jnp.float32)
    post_prob = jnp.sum(w * vals, axis=1)

    nearest = topi[:, 0]
    match = jnp.take(mem_values, nearest, axis=0) == label
    merged = q + jnp.take(mem_keys, nearest, axis=0)
    merged = merged / (jnp.linalg.norm(merged, axis=1, keepdims=True) + 1e-8)
    _, oldest = jax.lax.top_k(mem_ages, B)
    write_idx = jnp.where(match, nearest, oldest)
    write_key = jnp.where(match[:, None], merged, q)
    new_keys = new_keys_base.at[write_idx].set(write_key)
    new_values = mem_values.at[write_idx].set(label)
    new_ages = (mem_ages + 1.0).at[write_idx].set(0.0)
    return post_prob, new_keys, new_values, new_ages


# speedup vs baseline: 10.3961x; 10.3961x over previous
import functools

import jax
import jax.numpy as jnp
from jax.experimental import pallas as pl
from jax.experimental.pallas import tpu as pltpu

KEY_DIM = 128
MEM_SIZE = 65536
CHOOSE_K = 256
FC_DIM = 1024
BATCH = 256

MEM_CHUNK = 8192
N_CHUNKS = MEM_SIZE // MEM_CHUNK
GROUP = 16
N_GROUPS = MEM_SIZE // GROUP


def _q_kernel(x_ref, w1_ref, b1_ref, w2_ref, b2_ref, q_ref):
    h = jnp.maximum(
        jnp.dot(x_ref[...], w1_ref[...], preferred_element_type=jnp.float32)
        + b1_ref[...], 0.0)
    q = jnp.dot(h, w2_ref[...], preferred_element_type=jnp.float32) + b2_ref[...]
    qn = jnp.sqrt(jnp.sum(q * q, axis=1, keepdims=True))
    q_ref[...] = q / (qn + 1e-8)


def _sim_kernel(q_ref, mk_ref, sim_ref, nk_ref):
    nk_ref[...] = mk_ref[...]
    sim_ref[...] = jax.lax.dot_general(
        q_ref[...], mk_ref[...], (((1,), (1,)), ((), ())),
        preferred_element_type=jnp.float32)


def kernel(x, label, W1, b1, W2, b2, mem_keys, mem_values, mem_ages):
    B = x.shape[0]
    xf = x.reshape(B, -1)

    q = pl.pallas_call(
        _q_kernel,
        out_shape=jax.ShapeDtypeStruct((B, KEY_DIM), jnp.float32),
    )(xf, W1, b1.reshape(1, FC_DIM), W2, b2.reshape(1, KEY_DIM))

    sim, new_keys_base = pl.pallas_call(
        _sim_kernel,
        grid=(N_CHUNKS,),
        in_specs=[
            pl.BlockSpec((B, KEY_DIM), lambda i: (0, 0)),
            pl.BlockSpec((MEM_CHUNK, KEY_DIM), lambda i: (i, 0)),
        ],
        out_specs=[
            pl.BlockSpec((B, MEM_CHUNK), lambda i: (0, i)),
            pl.BlockSpec((MEM_CHUNK, KEY_DIM), lambda i: (i, 0)),
        ],
        out_shape=[
            jax.ShapeDtypeStruct((B, MEM_SIZE), jnp.float32),
            jax.ShapeDtypeStruct((MEM_SIZE, KEY_DIM), jnp.float32),
        ],
    )(q, mem_keys)

    gmax = jnp.max(sim.reshape(B, N_GROUPS, GROUP), axis=-1)
    _, gidx = jax.lax.top_k(gmax, CHOOSE_K)
    cand_idx = (gidx[:, :, None] * GROUP
                + jnp.arange(GROUP, dtype=gidx.dtype)).reshape(B, CHOOSE_K * GROUP)
    cand = jnp.take_along_axis(sim, cand_idx, axis=1)
    topv, topj = jax.lax.top_k(cand, CHOOSE_K)
    topi = jnp.take_along_axis(cand_idx, topj, axis=1)

    w = jax.nn.softmax(topv, axis=1)
    vals = jnp.take(mem_values, topi, axis=0).astype(jnp.float32)
    post_prob = jnp.sum(w * vals, axis=1)

    nearest = topi[:, 0]
    match = jnp.take(mem_values, nearest, axis=0) == label
    merged = q + jnp.take(mem_keys, nearest, axis=0)
    merged = merged / (jnp.linalg.norm(merged, axis=1, keepdims=True) + 1e-8)
    _, oldest = jax.lax.top_k(mem_ages, B)
    write_idx = jnp.where(match, nearest, oldest)
    write_key = jnp.where(match[:, None], merged, q)
    new_keys = new_keys_base.at[write_idx].set(write_key)
    new_values = mem_values.at[write_idx].set(label)
    new_ages = (mem_ages + 1.0).at[write_idx].set(0.0)
    return post_prob, new_keys, new_values, new_ages
